# baseline (speedup 1.0000x reference)
import functools

import jax
import jax.numpy as jnp
from jax import lax
from jax.experimental import pallas as pl
from jax.experimental.pallas import tpu as pltpu
from jax.experimental.pallas import tpu_sc as plsc

M = 2048
HIDDEN = 1024
INTER = 512
E = 64
CLAMP_LO = -10.0
CLAMP_HI = 10.0

TM = 64
NT = M // TM + E
P = NT * TM
GW = 128

NC = 2
NS = 16
NW = NC * NS
BPW = M // NW

@functools.lru_cache(maxsize=None)
def _sc_scatter_kernel():
    mesh = plsc.VectorSubcoreMesh(core_axis_name="c", subcore_axis_name="s")

    @functools.partial(
        pl.kernel,
        mesh=mesh,
        out_type=[
            jax.ShapeDtypeStruct((P, HIDDEN), jnp.float32),
            jax.ShapeDtypeStruct((P, GW), jnp.float32),
        ],
        scratch_types=[
            pltpu.VMEM((BPW,), jnp.int32),
            pltpu.VMEM((BPW, HIDDEN), jnp.float32),
            pltpu.VMEM((BPW, GW), jnp.float32),
            pltpu.SemaphoreType.DMA,
            pltpu.SemaphoreType.DMA,
        ],
    )
    def _sc_scatter(h_hbm, g_hbm, slot_hbm, px_hbm, pg_hbm,
                    idx_v, rows_v, grows_v, sem_x, sem_g):
        wid = lax.axis_index("s") * NC + lax.axis_index("c")
        base = wid * BPW
        pltpu.sync_copy(slot_hbm.at[pl.ds(base, BPW)], idx_v)
        pltpu.sync_copy(h_hbm.at[pl.ds(base, BPW)], rows_v)
        pltpu.sync_copy(g_hbm.at[pl.ds(base, BPW)], grows_v)
        cp_x = pltpu.async_copy(rows_v, px_hbm.at[idx_v], sem_x)
        cp_g = pltpu.async_copy(grows_v, pg_hbm.at[idx_v], sem_g)
        cp_x.wait()
        cp_g.wait()

    return _sc_scatter


@functools.lru_cache(maxsize=None)
def _sc_gather_kernel():
    mesh = plsc.VectorSubcoreMesh(core_axis_name="c", subcore_axis_name="s")

    @functools.partial(
        pl.kernel,
        mesh=mesh,
        out_type=jax.ShapeDtypeStruct((M, HIDDEN), jnp.float32),
        scratch_types=[
            pltpu.VMEM((BPW,), jnp.int32),
            pltpu.VMEM((BPW, HIDDEN), jnp.float32),
            pltpu.SemaphoreType.DMA,
        ],
    )
    def _sc_gather(py_hbm, slot_hbm, out_hbm, idx_v, rows_v, sem):
        wid = lax.axis_index("s") * NC + lax.axis_index("c")
        base = wid * BPW
        pltpu.sync_copy(slot_hbm.at[pl.ds(base, BPW)], idx_v)
        pltpu.async_copy(py_hbm.at[idx_v], rows_v, sem).wait()
        pltpu.sync_copy(rows_v, out_hbm.at[pl.ds(base, BPW)])

    return _sc_gather


def _gemm_outer(tg_ref, xm_ref, act_ref,
                x_hbm, gw_hbm, uw_hbm, dw_hbm, pg_hbm, y_hbm):
    def inner(idx, x_ref, gw_ref, uw_ref, dw_ref, pg_ref, y_ref):
        i = idx[0]

        @pl.when(act_ref[i] == 1)
        def _():
            _gemm_tile(x_ref, gw_ref, uw_ref, dw_ref, pg_ref, y_ref)

    wbuf = pl.Buffered(buffer_count=4, use_lookahead=True)
    xbuf = pl.Buffered(buffer_count=4, use_lookahead=True)
    pipe = pltpu.emit_pipeline(
        inner,
        grid=(NT,),
        in_specs=[
            pl.BlockSpec((TM, HIDDEN), lambda i: (xm_ref[i], 0),
                         pipeline_mode=xbuf),
            pl.BlockSpec((1, INTER, HIDDEN), lambda i: (tg_ref[i], 0, 0),
                         pipeline_mode=wbuf),
            pl.BlockSpec((1, INTER, HIDDEN), lambda i: (tg_ref[i], 0, 0),
                         pipeline_mode=wbuf),
            pl.BlockSpec((1, HIDDEN, INTER), lambda i: (tg_ref[i], 0, 0),
                         pipeline_mode=wbuf),
            pl.BlockSpec((TM, GW), lambda i: (xm_ref[i], 0),
                         pipeline_mode=xbuf),
        ],
        out_specs=[pl.BlockSpec((TM, HIDDEN), lambda i: (xm_ref[i], 0))],
        _explicit_indices=True,
    )
    pipe(x_hbm, gw_hbm, uw_hbm, dw_hbm, pg_hbm, y_hbm)


def _gemm_tile(x_ref, gw_ref, uw_ref, dw_ref, pg_ref, y_ref):
    x = x_ref[...]
    gw = gw_ref[0]
    uw = uw_ref[0]
    dn = (((1,), (1,)), ((), ()))
    g = lax.dot_general(x, gw, dn,
                        preferred_element_type=jnp.float32,
                        precision=lax.Precision.DEFAULT)
    g = jnp.minimum(g, CLAMP_HI)
    u = lax.dot_general(x, uw, dn,
                        preferred_element_type=jnp.float32,
                        precision=lax.Precision.DEFAULT)
    u = jnp.clip(u, CLAMP_LO, CLAMP_HI)
    sig = 1.0 / (1.0 + jnp.exp(-g))
    h = (g * sig) * u
    h = h * pg_ref[:, 0:1]
    dw = dw_ref[0]
    y = lax.dot_general(h, dw, dn,
                        preferred_element_type=jnp.float32,
                        precision=lax.Precision.DEFAULT)
    y_ref[...] = y


def _grouped_gemm(tg, xm, act, padded_x, gate_weight, up_weight,
                  down_weight, padded_g):
    smem = pl.BlockSpec(memory_space=pltpu.SMEM)
    hbm = pl.BlockSpec(memory_space=pltpu.MemorySpace.HBM)
    return pl.pallas_call(
        _gemm_outer,
        in_specs=[smem, smem, smem, hbm, hbm, hbm, hbm, hbm],
        out_specs=hbm,
        out_shape=jax.ShapeDtypeStruct((P, HIDDEN), jnp.float32),
    )(tg, xm, act, padded_x, gate_weight, up_weight, down_weight, padded_g)


_MC = M // 128


def _meta_body(e_ref, slot_ref, meta_ref, ranksel_ref):
    f32 = jnp.float32
    g_iota = lax.broadcasted_iota(jnp.int32, (E, 128), 0)
    g_col = lax.broadcasted_iota(jnp.int32, (E, 128), 0).astype(f32)
    io0 = lax.broadcasted_iota(jnp.int32, (128, 128), 0)
    io1 = lax.broadcasted_iota(jnp.int32, (128, 128), 1)
    a_up = (io0 < io1).astype(f32)
    ones128 = jnp.ones((128, 128), f32)
    jo0 = lax.broadcasted_iota(jnp.int32, (E, E), 0)
    jo1 = lax.broadcasted_iota(jnp.int32, (E, E), 1)
    l_incl = (jo1 <= jo0).astype(f32)
    dn = (((1,), (0,)), ((), ()))

    counts_b = jnp.zeros((E, 128), f32)
    for c in range(_MC):
        ohc = (g_iota == e_ref[c:c + 1, :]).astype(f32)
        ec = lax.dot_general(ohc, a_up, dn, preferred_element_type=f32)
        rank = ec + counts_b
        ranksel_ref[c:c + 1, :] = jnp.sum(ohc * rank, axis=0, keepdims=True)
        counts_b = counts_b + lax.dot_general(ohc, ones128, dn,
                                              preferred_element_type=f32)

    tiles_b = jnp.floor((counts_b + (TM - 1)) * (1.0 / TM))
    tile_cum_b = lax.dot_general(l_incl, tiles_b, dn,
                                 preferred_element_type=f32)
    start_b = tile_cum_b - tiles_b
    total_b = tile_cum_b[E - 1:E, :]

    ii64 = lax.broadcasted_iota(jnp.int32, (E, 128), 1).astype(f32)
    tg0 = jnp.sum((tile_cum_b <= ii64).astype(f32), axis=0, keepdims=True)
    lastg = jnp.max(g_col * (counts_b > 0).astype(f32), axis=0,
                    keepdims=True)
    ii1 = lax.broadcasted_iota(jnp.int32, (1, 128), 1).astype(f32)
    act1 = ii1 < total_b
    tgr = jnp.where(act1, tg0, lastg)
    xmr = jnp.where(act1, ii1, total_b - 1.0)
    actr = act1.astype(f32)
    ri = lax.broadcasted_iota(jnp.int32, (8, 128), 0)
    meta = jnp.where(ri == 0, tgr, jnp.where(ri == 1, xmr, actr))
    meta_ref[...] = meta.astype(jnp.int32)

    for c in range(_MC):
        ohc = (g_iota == e_ref[c:c + 1, :]).astype(f32)
        startsel = jnp.sum(ohc * start_b, axis=0, keepdims=True)
        slot_row = TM * startsel + ranksel_ref[c:c + 1, :]
        slot_ref[c:c + 1, :] = slot_row.astype(jnp.int32)


def _routing_metadata(e):
    e2d = e.reshape(_MC, 128)
    slot2d, meta = pl.pallas_call(
        _meta_body,
        out_shape=[
            jax.ShapeDtypeStruct((_MC, 128), jnp.int32),
            jax.ShapeDtypeStruct((8, 128), jnp.int32),
        ],
        scratch_shapes=[pltpu.VMEM((_MC, 128), jnp.float32)],
    )(e2d)
    slot = slot2d.reshape(M)
    tg = meta[0, :NT]
    xm = meta[1, :NT]
    act = meta[2, :NT]
    return slot, tg, xm, act


def kernel(flat_h, flat_idx, flat_gate, gate_weight, up_weight, down_weight):
    e = flat_idx[:, 0].astype(jnp.int32)
    slot, tg, xm, act = _routing_metadata(e)
    gate16 = jnp.broadcast_to(flat_gate.astype(jnp.float32), (M, GW))

    padded_x, padded_g = _sc_scatter_kernel()(flat_h, gate16, slot)
    padded_y = _grouped_gemm(tg, xm, act, padded_x, gate_weight, up_weight,
                             down_weight, padded_g)
    return _sc_gather_kernel()(padded_y, slot)


# speedup vs baseline: 1.0077x; 1.0077x over previous
import functools

import jax
import jax.numpy as jnp
from jax import lax
from jax.experimental import pallas as pl
from jax.experimental.pallas import tpu as pltpu
from jax.experimental.pallas import tpu_sc as plsc

M = 2048
HIDDEN = 1024
INTER = 512
E = 64
CLAMP_LO = -10.0
CLAMP_HI = 10.0

TM = 64
NT = M // TM + E
P = NT * TM
GW = 128

NC = 2
NS = 16
NW = NC * NS
BPW = M // NW

@functools.lru_cache(maxsize=None)
def _sc_scatter_kernel():
    mesh = plsc.VectorSubcoreMesh(core_axis_name="c", subcore_axis_name="s")

    @functools.partial(
        pl.kernel,
        mesh=mesh,
        out_type=[
            jax.ShapeDtypeStruct((P, HIDDEN), jnp.float32),
            jax.ShapeDtypeStruct((P, GW), jnp.float32),
        ],
        scratch_types=[
            pltpu.VMEM((BPW,), jnp.int32),
            pltpu.VMEM((BPW, HIDDEN), jnp.float32),
            pltpu.VMEM((BPW, GW), jnp.float32),
            pltpu.SemaphoreType.DMA,
            pltpu.SemaphoreType.DMA,
            pltpu.SemaphoreType.DMA,
        ],
    )
    def _sc_scatter(h_hbm, g_hbm, slot_hbm, px_hbm, pg_hbm,
                    idx_v, rows_v, grows_v, sem_x, sem_g, sem_i):
        wid = lax.axis_index("s") * NC + lax.axis_index("c")
        base = wid * BPW
        ld_i = pltpu.async_copy(slot_hbm.at[pl.ds(base, BPW)], idx_v, sem_i)
        ld_x = pltpu.async_copy(h_hbm.at[pl.ds(base, BPW)], rows_v, sem_x)
        ld_g = pltpu.async_copy(g_hbm.at[pl.ds(base, BPW)], grows_v, sem_g)
        ld_i.wait()
        ld_x.wait()
        ld_g.wait()
        cp_x = pltpu.async_copy(rows_v, px_hbm.at[idx_v], sem_x)
        cp_g = pltpu.async_copy(grows_v, pg_hbm.at[idx_v], sem_g)
        cp_x.wait()
        cp_g.wait()

    return _sc_scatter


@functools.lru_cache(maxsize=None)
def _sc_gather_kernel():
    mesh = plsc.VectorSubcoreMesh(core_axis_name="c", subcore_axis_name="s")

    @functools.partial(
        pl.kernel,
        mesh=mesh,
        out_type=jax.ShapeDtypeStruct((M, HIDDEN), jnp.float32),
        scratch_types=[
            pltpu.VMEM((BPW,), jnp.int32),
            pltpu.VMEM((BPW, HIDDEN), jnp.float32),
            pltpu.SemaphoreType.DMA,
        ],
    )
    def _sc_gather(py_hbm, slot_hbm, out_hbm, idx_v, rows_v, sem):
        wid = lax.axis_index("s") * NC + lax.axis_index("c")
        base = wid * BPW
        pltpu.sync_copy(slot_hbm.at[pl.ds(base, BPW)], idx_v)
        pltpu.async_copy(py_hbm.at[idx_v], rows_v, sem).wait()
        pltpu.sync_copy(rows_v, out_hbm.at[pl.ds(base, BPW)])

    return _sc_gather


def _gemm_outer(meta_ref,
                x_hbm, gw_hbm, uw_hbm, dw_hbm, pg_hbm, y_hbm):
    def inner(idx, x_ref, gw_ref, uw_ref, dw_ref, pg_ref, y_ref):
        i = idx[0]

        @pl.when(meta_ref[2, i] == 1)
        def _():
            _gemm_tile(x_ref, gw_ref, uw_ref, dw_ref, pg_ref, y_ref)

    wbuf = pl.Buffered(buffer_count=3, use_lookahead=True)
    xbuf = pl.Buffered(buffer_count=4, use_lookahead=True)
    pipe = pltpu.emit_pipeline(
        inner,
        grid=(NT,),
        in_specs=[
            pl.BlockSpec((TM, HIDDEN), lambda i: (meta_ref[1, i], 0),
                         pipeline_mode=xbuf),
            pl.BlockSpec((1, INTER, HIDDEN), lambda i: (meta_ref[0, i], 0, 0),
                         pipeline_mode=wbuf),
            pl.BlockSpec((1, INTER, HIDDEN), lambda i: (meta_ref[0, i], 0, 0),
                         pipeline_mode=wbuf),
            pl.BlockSpec((1, HIDDEN, INTER), lambda i: (meta_ref[0, i], 0, 0),
                         pipeline_mode=wbuf),
            pl.BlockSpec((TM, GW), lambda i: (meta_ref[1, i], 0),
                         pipeline_mode=xbuf),
        ],
        out_specs=[pl.BlockSpec((TM, HIDDEN), lambda i: (meta_ref[1, i], 0))],
        _explicit_indices=True,
    )
    pipe(x_hbm, gw_hbm, uw_hbm, dw_hbm, pg_hbm, y_hbm)


def _gemm_tile(x_ref, gw_ref, uw_ref, dw_ref, pg_ref, y_ref):
    x = x_ref[...]
    gw = gw_ref[0]
    uw = uw_ref[0]
    dn = (((1,), (1,)), ((), ()))
    g = lax.dot_general(x, gw, dn,
                        preferred_element_type=jnp.float32,
                        precision=lax.Precision.DEFAULT)
    g = jnp.minimum(g, CLAMP_HI)
    u = lax.dot_general(x, uw, dn,
                        preferred_element_type=jnp.float32,
                        precision=lax.Precision.DEFAULT)
    u = jnp.clip(u, CLAMP_LO, CLAMP_HI)
    sig = 1.0 / (1.0 + jnp.exp(-g))
    h = (g * sig) * u
    h = h * pg_ref[:, 0:1]
    dw = dw_ref[0]
    y = lax.dot_general(h, dw, dn,
                        preferred_element_type=jnp.float32,
                        precision=lax.Precision.DEFAULT)
    y_ref[...] = y


def _grouped_gemm(meta, padded_x, gate_weight, up_weight,
                  down_weight, padded_g):
    smem = pl.BlockSpec(memory_space=pltpu.SMEM)
    hbm = pl.BlockSpec(memory_space=pltpu.MemorySpace.HBM)
    return pl.pallas_call(
        _gemm_outer,
        in_specs=[smem, hbm, hbm, hbm, hbm, hbm],
        out_specs=hbm,
        out_shape=jax.ShapeDtypeStruct((P, HIDDEN), jnp.float32),
    )(meta, padded_x, gate_weight, up_weight, down_weight, padded_g)


_MC = M // 128


def _meta_body(e_ref, slot_ref, meta_ref, ranksel_ref):
    f32 = jnp.float32
    g_iota = lax.broadcasted_iota(jnp.int32, (E, 128), 0)
    g_col = lax.broadcasted_iota(jnp.int32, (E, 128), 0).astype(f32)
    io0 = lax.broadcasted_iota(jnp.int32, (128, 128), 0)
    io1 = lax.broadcasted_iota(jnp.int32, (128, 128), 1)
    a_up = (io0 < io1).astype(f32)
    ones128 = jnp.ones((128, 128), f32)
    jo0 = lax.broadcasted_iota(jnp.int32, (E, E), 0)
    jo1 = lax.broadcasted_iota(jnp.int32, (E, E), 1)
    l_incl = (jo1 <= jo0).astype(f32)
    dn = (((1,), (0,)), ((), ()))

    counts_b = jnp.zeros((E, 128), f32)
    for c in range(_MC):
        ohc = (g_iota == e_ref[c:c + 1, :]).astype(f32)
        ec = lax.dot_general(ohc, a_up, dn, preferred_element_type=f32)
        rank = ec + counts_b
        ranksel_ref[c:c + 1, :] = jnp.sum(ohc * rank, axis=0, keepdims=True)
        counts_b = counts_b + lax.dot_general(ohc, ones128, dn,
                                              preferred_element_type=f32)

    tiles_b = jnp.floor((counts_b + (TM - 1)) * (1.0 / TM))
    tile_cum_b = lax.dot_general(l_incl, tiles_b, dn,
                                 preferred_element_type=f32)
    start_b = tile_cum_b - tiles_b
    total_b = tile_cum_b[E - 1:E, :]

    ii64 = lax.broadcasted_iota(jnp.int32, (E, 128), 1).astype(f32)
    tg0 = jnp.sum((tile_cum_b <= ii64).astype(f32), axis=0, keepdims=True)
    lastg = jnp.max(g_col * (counts_b > 0).astype(f32), axis=0,
                    keepdims=True)
    ii1 = lax.broadcasted_iota(jnp.int32, (1, 128), 1).astype(f32)
    act1 = ii1 < total_b
    tgr = jnp.where(act1, tg0, lastg)
    xmr = jnp.where(act1, ii1, total_b - 1.0)
    actr = act1.astype(f32)
    ri = lax.broadcasted_iota(jnp.int32, (8, 128), 0)
    meta = jnp.where(ri == 0, tgr, jnp.where(ri == 1, xmr, actr))
    meta_ref[...] = meta.astype(jnp.int32)

    for c in range(_MC):
        ohc = (g_iota == e_ref[c:c + 1, :]).astype(f32)
        startsel = jnp.sum(ohc * start_b, axis=0, keepdims=True)
        slot_row = TM * startsel + ranksel_ref[c:c + 1, :]
        slot_ref[c:c + 1, :] = slot_row.astype(jnp.int32)


def _routing_metadata(e):
    e2d = e.reshape(_MC, 128)
    slot2d, meta = pl.pallas_call(
        _meta_body,
        out_shape=[
            jax.ShapeDtypeStruct((_MC, 128), jnp.int32),
            jax.ShapeDtypeStruct((8, 128), jnp.int32),
        ],
        scratch_shapes=[pltpu.VMEM((_MC, 128), jnp.float32)],
    )(e2d)
    return slot2d.reshape(M), meta


def kernel(flat_h, flat_idx, flat_gate, gate_weight, up_weight, down_weight):
    e = flat_idx[:, 0].astype(jnp.int32)
    slot, meta = _routing_metadata(e)
    gate16 = jnp.broadcast_to(flat_gate.astype(jnp.float32), (M, GW))

    padded_x, padded_g = _sc_scatter_kernel()(flat_h, gate16, slot)
    padded_y = _grouped_gemm(meta, padded_x, gate_weight, up_weight,
                             down_weight, padded_g)
    return _sc_gather_kernel()(padded_y, slot)


# speedup vs baseline: 1.0157x; 1.0079x over previous
import functools

import jax
import jax.numpy as jnp
from jax import lax
from jax.experimental import pallas as pl
from jax.experimental.pallas import tpu as pltpu
from jax.experimental.pallas import tpu_sc as plsc

M = 2048
HIDDEN = 1024
INTER = 512
E = 64
CLAMP_LO = -10.0
CLAMP_HI = 10.0

TM = 64
NT = M // TM + E
P = NT * TM
GW = 128

NC = 2
NS = 16
NW = NC * NS
BPW = M // NW

@functools.lru_cache(maxsize=None)
def _sc_scatter_kernel():
    mesh = plsc.VectorSubcoreMesh(core_axis_name="c", subcore_axis_name="s")

    @functools.partial(
        pl.kernel,
        mesh=mesh,
        out_type=[
            jax.ShapeDtypeStruct((P, HIDDEN), jnp.float32),
            jax.ShapeDtypeStruct((P, GW), jnp.float32),
        ],
        scratch_types=[
            pltpu.VMEM((BPW,), jnp.int32),
            pltpu.VMEM((BPW, HIDDEN), jnp.float32),
            pltpu.VMEM((BPW, GW), jnp.float32),
            pltpu.SemaphoreType.DMA,
            pltpu.SemaphoreType.DMA,
            pltpu.SemaphoreType.DMA,
        ],
    )
    def _sc_scatter(h_hbm, g_hbm, slot_hbm, px_hbm, pg_hbm,
                    idx_v, rows_v, grows_v, sem_x, sem_g, sem_i):
        wid = lax.axis_index("s") * NC + lax.axis_index("c")
        base = wid * BPW
        ld_i = pltpu.async_copy(slot_hbm.at[pl.ds(base, BPW)], idx_v, sem_i)
        ld_x = pltpu.async_copy(h_hbm.at[pl.ds(base, BPW)], rows_v, sem_x)
        ld_g = pltpu.async_copy(g_hbm.at[pl.ds(base, BPW)], grows_v, sem_g)
        ld_i.wait()
        ld_x.wait()
        ld_g.wait()
        cp_x = pltpu.async_copy(rows_v, px_hbm.at[idx_v], sem_x)
        cp_g = pltpu.async_copy(grows_v, pg_hbm.at[idx_v], sem_g)
        cp_x.wait()
        cp_g.wait()

    return _sc_scatter


@functools.lru_cache(maxsize=None)
def _sc_gather_kernel():
    mesh = plsc.VectorSubcoreMesh(core_axis_name="c", subcore_axis_name="s")

    @functools.partial(
        pl.kernel,
        mesh=mesh,
        out_type=jax.ShapeDtypeStruct((M, HIDDEN), jnp.float32),
        scratch_types=[
            pltpu.VMEM((BPW,), jnp.int32),
            pltpu.VMEM((BPW, HIDDEN), jnp.float32),
            pltpu.SemaphoreType.DMA,
        ],
    )
    def _sc_gather(py_hbm, slot_hbm, out_hbm, idx_v, rows_v, sem):
        wid = lax.axis_index("s") * NC + lax.axis_index("c")
        base = wid * BPW
        pltpu.sync_copy(slot_hbm.at[pl.ds(base, BPW)], idx_v)
        pltpu.async_copy(py_hbm.at[idx_v], rows_v, sem).wait()
        pltpu.sync_copy(rows_v, out_hbm.at[pl.ds(base, BPW)])

    return _sc_gather


def _gemm_outer(meta_ref,
                x_hbm, gw_hbm, uw_hbm, dw_hbm, pg_hbm, y_hbm):
    def inner(idx, x_ref, gw_ref, uw_ref, dw_ref, pg_ref, y_ref):
        i = idx[0]

        @pl.when(meta_ref[2, i] == 1)
        def _():
            _gemm_tile(x_ref, gw_ref, uw_ref, dw_ref, pg_ref, y_ref)

    wbuf = pl.Buffered(buffer_count=3, use_lookahead=True)
    xbuf = pl.Buffered(buffer_count=4, use_lookahead=True)
    pipe = pltpu.emit_pipeline(
        inner,
        grid=(NT,),
        in_specs=[
            pl.BlockSpec((TM, HIDDEN), lambda i: (meta_ref[1, i], 0),
                         pipeline_mode=xbuf),
            pl.BlockSpec((1, INTER, HIDDEN), lambda i: (meta_ref[0, i], 0, 0),
                         pipeline_mode=wbuf),
            pl.BlockSpec((1, INTER, HIDDEN), lambda i: (meta_ref[0, i], 0, 0),
                         pipeline_mode=wbuf),
            pl.BlockSpec((1, HIDDEN, INTER), lambda i: (meta_ref[0, i], 0, 0),
                         pipeline_mode=wbuf),
            pl.BlockSpec((TM, GW), lambda i: (meta_ref[1, i], 0),
                         pipeline_mode=xbuf),
        ],
        out_specs=[pl.BlockSpec((TM, HIDDEN), lambda i: (meta_ref[1, i], 0))],
        _explicit_indices=True,
    )
    pipe(x_hbm, gw_hbm, uw_hbm, dw_hbm, pg_hbm, y_hbm)


def _gemm_tile(x_ref, gw_ref, uw_ref, dw_ref, pg_ref, y_ref):
    x = x_ref[...]
    gw = gw_ref[0]
    uw = uw_ref[0]
    dn = (((1,), (1,)), ((), ()))
    g = lax.dot_general(x, gw, dn,
                        preferred_element_type=jnp.float32,
                        precision=lax.Precision.DEFAULT)
    g = jnp.minimum(g, CLAMP_HI)
    u = lax.dot_general(x, uw, dn,
                        preferred_element_type=jnp.float32,
                        precision=lax.Precision.DEFAULT)
    u = jnp.clip(u, CLAMP_LO, CLAMP_HI)
    sig = 1.0 / (1.0 + jnp.exp(-g))
    h = (g * sig) * u
    h = h * pg_ref[:, 0:1]
    dw = dw_ref[0]
    y = lax.dot_general(h, dw, dn,
                        preferred_element_type=jnp.float32,
                        precision=lax.Precision.DEFAULT)
    y_ref[...] = y


def _grouped_gemm(meta, padded_x, gate_weight, up_weight,
                  down_weight, padded_g):
    smem = pl.BlockSpec(memory_space=pltpu.SMEM)
    hbm = pl.BlockSpec(memory_space=pltpu.MemorySpace.HBM)
    return pl.pallas_call(
        _gemm_outer,
        in_specs=[smem, hbm, hbm, hbm, hbm, hbm],
        out_specs=hbm,
        out_shape=jax.ShapeDtypeStruct((P, HIDDEN), jnp.float32),
    )(meta, padded_x, gate_weight, up_weight, down_weight, padded_g)


_MC = M // 128


def _meta_body(e_ref, gate_ref, slot_ref, meta_ref, gate16_ref, ranksel_ref):
    f32 = jnp.float32
    g_iota = lax.broadcasted_iota(jnp.int32, (E, 128), 0)
    g_col = lax.broadcasted_iota(jnp.int32, (E, 128), 0).astype(f32)
    io0 = lax.broadcasted_iota(jnp.int32, (128, 128), 0)
    io1 = lax.broadcasted_iota(jnp.int32, (128, 128), 1)
    a_up = (io0 < io1).astype(f32)
    ones128 = jnp.ones((128, 128), f32)
    jo0 = lax.broadcasted_iota(jnp.int32, (E, E), 0)
    jo1 = lax.broadcasted_iota(jnp.int32, (E, E), 1)
    l_incl = (jo1 <= jo0).astype(f32)
    dn = (((1,), (0,)), ((), ()))

    counts_b = jnp.zeros((E, 128), f32)
    for c in range(_MC):
        ohc = (g_iota == e_ref[c:c + 1, :]).astype(f32)
        ec = lax.dot_general(ohc, a_up, dn, preferred_element_type=f32)
        rank = ec + counts_b
        ranksel_ref[c:c + 1, :] = jnp.sum(ohc * rank, axis=0, keepdims=True)
        counts_b = counts_b + lax.dot_general(ohc, ones128, dn,
                                              preferred_element_type=f32)

    tiles_b = jnp.floor((counts_b + (TM - 1)) * (1.0 / TM))
    tile_cum_b = lax.dot_general(l_incl, tiles_b, dn,
                                 preferred_element_type=f32)
    start_b = tile_cum_b - tiles_b
    total_b = tile_cum_b[E - 1:E, :]

    ii64 = lax.broadcasted_iota(jnp.int32, (E, 128), 1).astype(f32)
    tg0 = jnp.sum((tile_cum_b <= ii64).astype(f32), axis=0, keepdims=True)
    lastg = jnp.max(g_col * (counts_b > 0).astype(f32), axis=0,
                    keepdims=True)
    ii1 = lax.broadcasted_iota(jnp.int32, (1, 128), 1).astype(f32)
    act1 = ii1 < total_b
    tgr = jnp.where(act1, tg0, lastg)
    xmr = jnp.where(act1, ii1, total_b - 1.0)
    actr = act1.astype(f32)
    ri = lax.broadcasted_iota(jnp.int32, (8, 128), 0)
    meta = jnp.where(ri == 0, tgr, jnp.where(ri == 1, xmr, actr))
    meta_ref[...] = meta.astype(jnp.int32)

    diag = (io0 == io1).astype(f32)
    for c in range(_MC):
        ohc = (g_iota == e_ref[c:c + 1, :]).astype(f32)
        startsel = jnp.sum(ohc * start_b, axis=0, keepdims=True)
        slot_row = TM * startsel + ranksel_ref[c:c + 1, :]
        slot_ref[c:c + 1, :] = slot_row.astype(jnp.int32)
        dg = diag * gate_ref[c:c + 1, :]
        gate16_ref[c * 128:(c + 1) * 128, :] = lax.dot_general(
            dg, ones128, dn, preferred_element_type=f32)[:, :GW]


def _routing_metadata(e, gate):
    e2d = e.reshape(_MC, 128)
    g2d = gate.reshape(_MC, 128)
    slot2d, meta, gate16 = pl.pallas_call(
        _meta_body,
        out_shape=[
            jax.ShapeDtypeStruct((_MC, 128), jnp.int32),
            jax.ShapeDtypeStruct((8, 128), jnp.int32),
            jax.ShapeDtypeStruct((M, GW), jnp.float32),
        ],
        scratch_shapes=[pltpu.VMEM((_MC, 128), jnp.float32)],
    )(e2d, g2d)
    return slot2d.reshape(M), meta, gate16


def kernel(flat_h, flat_idx, flat_gate, gate_weight, up_weight, down_weight):
    e = flat_idx[:, 0].astype(jnp.int32)
    slot, meta, gate16 = _routing_metadata(e, flat_gate.reshape(M))

    padded_x, padded_g = _sc_scatter_kernel()(flat_h, gate16, slot)
    padded_y = _grouped_gemm(meta, padded_x, gate_weight, up_weight,
                             down_weight, padded_g)
    return _sc_gather_kernel()(padded_y, slot)


# speedup vs baseline: 1.0182x; 1.0025x over previous
import functools

import jax
import jax.numpy as jnp
from jax import lax
from jax.experimental import pallas as pl
from jax.experimental.pallas import tpu as pltpu
from jax.experimental.pallas import tpu_sc as plsc

M = 2048
HIDDEN = 1024
INTER = 512
E = 64
CLAMP_LO = -10.0
CLAMP_HI = 10.0

TM = 64
NT = M // TM + E
P = NT * TM
GW = 128

NC = 2
NS = 16
NW = NC * NS
BPW = M // NW

@functools.lru_cache(maxsize=None)
def _sc_scatter_kernel():
    mesh = plsc.VectorSubcoreMesh(core_axis_name="c", subcore_axis_name="s")

    @functools.partial(
        pl.kernel,
        mesh=mesh,
        out_type=[
            jax.ShapeDtypeStruct((P, HIDDEN), jnp.float32),
            jax.ShapeDtypeStruct((P, GW), jnp.float32),
        ],
        scratch_types=[
            pltpu.VMEM((BPW,), jnp.int32),
            pltpu.VMEM((BPW, HIDDEN), jnp.float32),
            pltpu.VMEM((BPW, GW), jnp.float32),
            pltpu.SemaphoreType.DMA,
            pltpu.SemaphoreType.DMA,
            pltpu.SemaphoreType.DMA,
        ],
    )
    def _sc_scatter(h_hbm, g_hbm, slot_hbm, px_hbm, pg_hbm,
                    idx_v, rows_v, grows_v, sem_x, sem_g, sem_i):
        wid = lax.axis_index("s") * NC + lax.axis_index("c")
        base = wid * BPW
        ld_i = pltpu.async_copy(slot_hbm.at[pl.ds(base, BPW)], idx_v, sem_i)
        ld_x = pltpu.async_copy(h_hbm.at[pl.ds(base, BPW)], rows_v, sem_x)
        ld_g = pltpu.async_copy(g_hbm.at[pl.ds(base, BPW)], grows_v, sem_g)
        ld_i.wait()
        ld_x.wait()
        ld_g.wait()
        cp_x = pltpu.async_copy(rows_v, px_hbm.at[idx_v], sem_x)
        cp_g = pltpu.async_copy(grows_v, pg_hbm.at[idx_v], sem_g)
        cp_x.wait()
        cp_g.wait()

    return _sc_scatter


@functools.lru_cache(maxsize=None)
def _sc_gather_kernel():
    mesh = plsc.VectorSubcoreMesh(core_axis_name="c", subcore_axis_name="s")

    @functools.partial(
        pl.kernel,
        mesh=mesh,
        out_type=jax.ShapeDtypeStruct((M, HIDDEN), jnp.float32),
        scratch_types=[
            pltpu.VMEM((BPW,), jnp.int32),
            pltpu.VMEM((BPW, HIDDEN), jnp.float32),
            pltpu.SemaphoreType.DMA,
        ],
    )
    def _sc_gather(py_hbm, slot_hbm, out_hbm, idx_v, rows_v, sem):
        wid = lax.axis_index("s") * NC + lax.axis_index("c")
        base = wid * BPW
        pltpu.sync_copy(slot_hbm.at[pl.ds(base, BPW)], idx_v)
        pltpu.async_copy(py_hbm.at[idx_v], rows_v, sem).wait()
        pltpu.sync_copy(rows_v, out_hbm.at[pl.ds(base, BPW)])

    return _sc_gather


def _gemm_outer(meta_ref,
                x_hbm, gw_hbm, uw_hbm, dw_hbm, pg_hbm, y_hbm):
    def inner(idx, x_ref, gw_ref, uw_ref, dw_ref, pg_ref, y_ref):
        i = idx[0]

        @pl.when(meta_ref[2, i] == 1)
        def _():
            _gemm_tile(x_ref, gw_ref, uw_ref, dw_ref, pg_ref, y_ref)

    wbuf = pl.Buffered(buffer_count=3, use_lookahead=True)
    xbuf = pl.Buffered(buffer_count=8, use_lookahead=True)
    pipe = pltpu.emit_pipeline(
        inner,
        grid=(NT,),
        in_specs=[
            pl.BlockSpec((TM, HIDDEN), lambda i: (meta_ref[1, i], 0),
                         pipeline_mode=xbuf),
            pl.BlockSpec((1, INTER, HIDDEN), lambda i: (meta_ref[0, i], 0, 0),
                         pipeline_mode=wbuf),
            pl.BlockSpec((1, INTER, HIDDEN), lambda i: (meta_ref[0, i], 0, 0),
                         pipeline_mode=wbuf),
            pl.BlockSpec((1, HIDDEN, INTER), lambda i: (meta_ref[0, i], 0, 0),
                         pipeline_mode=wbuf),
            pl.BlockSpec((TM, GW), lambda i: (meta_ref[1, i], 0),
                         pipeline_mode=xbuf),
        ],
        out_specs=[pl.BlockSpec((TM, HIDDEN), lambda i: (meta_ref[1, i], 0))],
        _explicit_indices=True,
    )
    pipe(x_hbm, gw_hbm, uw_hbm, dw_hbm, pg_hbm, y_hbm)


def _gemm_tile(x_ref, gw_ref, uw_ref, dw_ref, pg_ref, y_ref):
    x = x_ref[...]
    gw = gw_ref[0]
    uw = uw_ref[0]
    dn = (((1,), (1,)), ((), ()))
    g = lax.dot_general(x, gw, dn,
                        preferred_element_type=jnp.float32,
                        precision=lax.Precision.DEFAULT)
    g = jnp.minimum(g, CLAMP_HI)
    u = lax.dot_general(x, uw, dn,
                        preferred_element_type=jnp.float32,
                        precision=lax.Precision.DEFAULT)
    u = jnp.clip(u, CLAMP_LO, CLAMP_HI)
    sig = 1.0 / (1.0 + jnp.exp(-g))
    h = (g * sig) * u
    h = h * pg_ref[:, 0:1]
    dw = dw_ref[0]
    y = lax.dot_general(h, dw, dn,
                        preferred_element_type=jnp.float32,
                        precision=lax.Precision.DEFAULT)
    y_ref[...] = y


def _grouped_gemm(meta, padded_x, gate_weight, up_weight,
                  down_weight, padded_g):
    smem = pl.BlockSpec(memory_space=pltpu.SMEM)
    hbm = pl.BlockSpec(memory_space=pltpu.MemorySpace.HBM)
    return pl.pallas_call(
        _gemm_outer,
        in_specs=[smem, hbm, hbm, hbm, hbm, hbm],
        out_specs=hbm,
        out_shape=jax.ShapeDtypeStruct((P, HIDDEN), jnp.float32),
    )(meta, padded_x, gate_weight, up_weight, down_weight, padded_g)


_MC = M // 128


def _meta_body(e_ref, gate_ref, slot_ref, meta_ref, gate16_ref, ranksel_ref):
    f32 = jnp.float32
    g_iota = lax.broadcasted_iota(jnp.int32, (E, 128), 0)
    g_col = lax.broadcasted_iota(jnp.int32, (E, 128), 0).astype(f32)
    io0 = lax.broadcasted_iota(jnp.int32, (128, 128), 0)
    io1 = lax.broadcasted_iota(jnp.int32, (128, 128), 1)
    a_up = (io0 < io1).astype(f32)
    ones128 = jnp.ones((128, 128), f32)
    jo0 = lax.broadcasted_iota(jnp.int32, (E, E), 0)
    jo1 = lax.broadcasted_iota(jnp.int32, (E, E), 1)
    l_incl = (jo1 <= jo0).astype(f32)
    dn = (((1,), (0,)), ((), ()))

    counts_b = jnp.zeros((E, 128), f32)
    for c in range(_MC):
        ohc = (g_iota == e_ref[c:c + 1, :]).astype(f32)
        ec = lax.dot_general(ohc, a_up, dn, preferred_element_type=f32)
        rank = ec + counts_b
        ranksel_ref[c:c + 1, :] = jnp.sum(ohc * rank, axis=0, keepdims=True)
        counts_b = counts_b + lax.dot_general(ohc, ones128, dn,
                                              preferred_element_type=f32)

    tiles_b = jnp.floor((counts_b + (TM - 1)) * (1.0 / TM))
    tile_cum_b = lax.dot_general(l_incl, tiles_b, dn,
                                 preferred_element_type=f32)
    start_b = tile_cum_b - tiles_b
    total_b = tile_cum_b[E - 1:E, :]

    ii64 = lax.broadcasted_iota(jnp.int32, (E, 128), 1).astype(f32)
    tg0 = jnp.sum((tile_cum_b <= ii64).astype(f32), axis=0, keepdims=True)
    lastg = jnp.max(g_col * (counts_b > 0).astype(f32), axis=0,
                    keepdims=True)
    ii1 = lax.broadcasted_iota(jnp.int32, (1, 128), 1).astype(f32)
    act1 = ii1 < total_b
    tgr = jnp.where(act1, tg0, lastg)
    xmr = jnp.where(act1, ii1, total_b - 1.0)
    actr = act1.astype(f32)
    ri = lax.broadcasted_iota(jnp.int32, (8, 128), 0)
    meta = jnp.where(ri == 0, tgr, jnp.where(ri == 1, xmr, actr))
    meta_ref[...] = meta.astype(jnp.int32)

    diag = (io0 == io1).astype(f32)
    for c in range(_MC):
        ohc = (g_iota == e_ref[c:c + 1, :]).astype(f32)
        startsel = jnp.sum(ohc * start_b, axis=0, keepdims=True)
        slot_row = TM * startsel + ranksel_ref[c:c + 1, :]
        slot_ref[c:c + 1, :] = slot_row.astype(jnp.int32)
        dg = diag * gate_ref[c:c + 1, :]
        gate16_ref[c * 128:(c + 1) * 128, :] = lax.dot_general(
            dg, ones128, dn, preferred_element_type=f32)[:, :GW]


def _routing_metadata(e, gate):
    e2d = e.reshape(_MC, 128)
    g2d = gate.reshape(_MC, 128)
    slot2d, meta, gate16 = pl.pallas_call(
        _meta_body,
        out_shape=[
            jax.ShapeDtypeStruct((_MC, 128), jnp.int32),
            jax.ShapeDtypeStruct((8, 128), jnp.int32),
            jax.ShapeDtypeStruct((M, GW), jnp.float32),
        ],
        scratch_shapes=[pltpu.VMEM((_MC, 128), jnp.float32)],
    )(e2d, g2d)
    return slot2d.reshape(M), meta, gate16


def kernel(flat_h, flat_idx, flat_gate, gate_weight, up_weight, down_weight):
    e = flat_idx[:, 0].astype(jnp.int32)
    slot, meta, gate16 = _routing_metadata(e, flat_gate.reshape(M))

    padded_x, padded_g = _sc_scatter_kernel()(flat_h, gate16, slot)
    padded_y = _grouped_gemm(meta, padded_x, gate_weight, up_weight,
                             down_weight, padded_g)
    return _sc_gather_kernel()(padded_y, slot)
